# Initial kernel scaffold; baseline (speedup 1.0000x reference)
import functools

import jax
import jax.numpy as jnp
from jax import lax
from jax.experimental import pallas as pl
from jax.experimental.pallas import tpu as pltpu
from jax.experimental.pallas import tpu_sc as plsc

N = 10000
D = 128
EDGES = 320000
N_BLK = 1000
GRID = N // N_BLK

NC = 2
NS = 16
CHUNK = 80
EPW = EDGES // (NC * NS)
NCHUNK = EPW // CHUNK
ROWS_PER_SUB = N // NS
ZROWS = 125


def _emb_body(x_ref, w_ref, b_ref, o_ref):
    o_ref[...] = (
        jnp.dot(x_ref[...], w_ref[...], preferred_element_type=jnp.float32)
        + b_ref[...]
    )


def _ab_body(h_ref, w_ref, b_ref, ax_ref, bx_ref):
    ab = (
        jnp.dot(h_ref[...], w_ref[...], preferred_element_type=jnp.float32)
        + b_ref[...]
    )
    ax_ref[...] = ab[:, :D]
    bx_ref[...] = ab[:, D:]


def _t_body(ax_ref, p_ref, sn_ref, t_ref, ps_ref, pq_ref):
    i = pl.program_id(0)
    t = (ax_ref[...] + p_ref[0] + p_ref[1]) * sn_ref[...]
    t_ref[...] = t
    s = jnp.sum(t.reshape(N_BLK // 8, 8, D), axis=0)
    q = jnp.sum((t * t).reshape(N_BLK // 8, 8, D), axis=0)

    @pl.when(i == 0)
    def _():
        ps_ref[...] = jnp.zeros_like(ps_ref)
        pq_ref[...] = jnp.zeros_like(pq_ref)

    ps_ref[...] += s
    pq_ref[...] += q


def _bn_body(t_ref, h_ref, ps_ref, pq_ref, g_ref, b_ref, o_ref, hs_ref):
    i = pl.program_id(0)
    mu = jnp.sum(ps_ref[...], axis=0, keepdims=True) * (1.0 / N)
    var = jnp.sum(pq_ref[...], axis=0, keepdims=True) * (1.0 / N) - mu * mu
    inv = lax.rsqrt(var + 1e-5)
    t = t_ref[...]
    hn = h_ref[...] + jnp.maximum(g_ref[...] * (t - mu) * inv + b_ref[...], 0.0)
    o_ref[...] = hn

    @pl.when(i == 0)
    def _():
        hs_ref[...] = jnp.zeros_like(hs_ref)

    hs_ref[...] += jnp.sum(hn.reshape(N_BLK // 8, 8, D), axis=0)


def _mlp_body(hs_ref, w0_ref, b0_ref, w1_ref, b1_ref, w2_ref, b2_ref, o_ref):
    m = jnp.sum(hs_ref[...], axis=0, keepdims=True) * (1.0 / N)
    y = jnp.maximum(
        jnp.dot(m, w0_ref[...], preferred_element_type=jnp.float32) + b0_ref[...], 0.0)
    y = jnp.maximum(
        jnp.dot(y, w1_ref[...], preferred_element_type=jnp.float32) + b1_ref[...], 0.0)
    o_ref[...] = (
        jnp.dot(y, w2_ref[...], preferred_element_type=jnp.float32) + b2_ref[...]
    )


_row_spec = pl.BlockSpec((N_BLK, D), lambda i: (i, 0))
_full_w = pl.BlockSpec((D, D), lambda i: (0, 0))
_full_b = pl.BlockSpec((1, D), lambda i: (0, 0))
_acc_spec = pl.BlockSpec((8, D), lambda i: (0, 0))


def _emb(x, w, b):
    return pl.pallas_call(
        _emb_body,
        grid=(GRID,),
        in_specs=[_row_spec, _full_w, _full_b],
        out_specs=_row_spec,
        out_shape=jax.ShapeDtypeStruct((N, D), jnp.float32),
    )(x, w, b)


def _ab(h, wcat, bcat):
    return pl.pallas_call(
        _ab_body,
        grid=(GRID,),
        in_specs=[
            _row_spec,
            pl.BlockSpec((D, 2 * D), lambda i: (0, 0)),
            pl.BlockSpec((1, 2 * D), lambda i: (0, 0)),
        ],
        out_specs=[_row_spec, _row_spec],
        out_shape=[
            jax.ShapeDtypeStruct((N, D), jnp.float32),
            jax.ShapeDtypeStruct((N, D), jnp.float32),
        ],
    )(h, wcat, bcat)


def _t_stage(ax, p, snorm):
    return pl.pallas_call(
        _t_body,
        grid=(GRID,),
        in_specs=[
            _row_spec,
            pl.BlockSpec((2, N_BLK, D), lambda i: (0, i, 0)),
            pl.BlockSpec((N_BLK, 1), lambda i: (i, 0)),
        ],
        out_specs=[_row_spec, _acc_spec, _acc_spec],
        out_shape=[
            jax.ShapeDtypeStruct((N, D), jnp.float32),
            jax.ShapeDtypeStruct((8, D), jnp.float32),
            jax.ShapeDtypeStruct((8, D), jnp.float32),
        ],
    )(ax, p, snorm)


def _bn_stage(t, h, ps, pq, g, b):
    return pl.pallas_call(
        _bn_body,
        grid=(GRID,),
        in_specs=[_row_spec, _row_spec, _acc_spec, _acc_spec, _full_b, _full_b],
        out_specs=[_row_spec, _acc_spec],
        out_shape=[
            jax.ShapeDtypeStruct((N, D), jnp.float32),
            jax.ShapeDtypeStruct((8, D), jnp.float32),
        ],
    )(t, h, ps, pq, g, b)


def _mlp(hs, w0, b0, w1, b1, w2, b2):
    return pl.pallas_call(
        _mlp_body,
        grid=(1,),
        in_specs=[_acc_spec, _full_w, _full_b, _full_w, _full_b, _full_w, _full_b],
        out_specs=pl.BlockSpec((1, D), lambda i: (0, 0)),
        out_shape=jax.ShapeDtypeStruct((1, D), jnp.float32),
    )(hs, w0, b0, w1, b1, w2, b2)


def _sc_agg_kernel(bx_hbm, src_hbm, dst_hbm, out_hbm,
                   src_v, dst_v, rows_v, zbuf_v, acc_sh, sem):
    cid = lax.axis_index("c")
    sid = lax.axis_index("s")

    @pl.loop(0, ZROWS)
    def _(r):
        @pl.loop(0, D, step=16)
        def _(j):
            zbuf_v[r, pl.ds(j, 16)] = jnp.zeros((16,), jnp.float32)

    @pl.loop(0, ROWS_PER_SUB, step=ZROWS)
    def _(r):
        pltpu.sync_copy(zbuf_v, acc_sh.at[pl.ds(sid * ROWS_PER_SUB + r, ZROWS)])

    plsc.subcore_barrier()

    base = cid * (EDGES // NC) + sid * EPW

    @pl.loop(0, NCHUNK)
    def _(t):
        off = base + t * CHUNK
        pltpu.sync_copy(src_hbm.at[pl.ds(off, CHUNK)], src_v)
        pltpu.sync_copy(dst_hbm.at[pl.ds(off, CHUNK)], dst_v)
        pltpu.async_copy(bx_hbm.at[src_v], rows_v, sem).wait()
        pltpu.sync_copy(rows_v, acc_sh.at[dst_v], add=True)

    plsc.subcore_barrier()

    @pl.loop(0, ROWS_PER_SUB, step=ZROWS)
    def _(r):
        r0 = sid * ROWS_PER_SUB + r
        pltpu.sync_copy(acc_sh.at[pl.ds(r0, ZROWS)],
                        out_hbm.at[cid].at[pl.ds(r0, ZROWS)])


@jax.jit
def _sc_agg(bx, src, dst):
    mesh = plsc.VectorSubcoreMesh(core_axis_name="c", subcore_axis_name="s")
    f = pl.kernel(
        _sc_agg_kernel,
        mesh=mesh,
        out_type=jax.ShapeDtypeStruct((NC, N, D), jnp.float32),
        scratch_types=[
            pltpu.VMEM((CHUNK,), jnp.int32),
            pltpu.VMEM((CHUNK,), jnp.int32),
            pltpu.VMEM((CHUNK, D), jnp.float32),
            pltpu.VMEM((ZROWS, D), jnp.float32),
            pltpu.VMEM_SHARED((N, D), jnp.float32),
            pltpu.SemaphoreType.DMA,
        ],
    )
    return f(bx, src, dst)


def kernel(X, E, snorm_n, snorm_e, params, edge_index):
    src = edge_index[0]
    dst = edge_index[1]

    H = _emb(X, params['emb_h_w'], params['emb_h_b'].reshape(1, D))

    for lp in params['layers']:
        wcat = jnp.concatenate([lp['A_w'], lp['B_w']], axis=1)
        bcat = jnp.concatenate([lp['A_b'], lp['B_b']]).reshape(1, 2 * D)
        ax, bx = _ab(H, wcat, bcat)
        p = _sc_agg(bx, src, dst)
        t, ps, pq = _t_stage(ax, p, snorm_n)
        H, hs = _bn_stage(t, H, ps, pq,
                          lp['bn_g'].reshape(1, D), lp['bn_b'].reshape(1, D))

    mlp = params['mlp']
    return _mlp(hs, mlp[0]['w'], mlp[0]['b'].reshape(1, D),
                mlp[1]['w'], mlp[1]['b'].reshape(1, D),
                mlp[2]['w'], mlp[2]['b'].reshape(1, D))


# speedup vs baseline: 4.7095x; 4.7095x over previous
import functools

import jax
import jax.numpy as jnp
from jax import lax
from jax.experimental import pallas as pl
from jax.experimental.pallas import tpu as pltpu
from jax.experimental.pallas import tpu_sc as plsc

N = 10000
D = 128
EDGES = 320000
N_BLK = 1000
GRID = N // N_BLK

NC = 2
NS = 16
CHUNK = 80
EPW = EDGES // (NC * NS)
NCHUNK = EPW // CHUNK
DCHUNK = 200


def _emb_body(x_ref, w_ref, b_ref, o_ref):
    o_ref[...] = (
        jnp.dot(x_ref[...], w_ref[...], preferred_element_type=jnp.float32)
        + b_ref[...]
    )


def _ab_body(h_ref, w_ref, b_ref, ax_ref, bx_ref):
    ab = (
        jnp.dot(h_ref[...], w_ref[...], preferred_element_type=jnp.float32)
        + b_ref[...]
    )
    ax_ref[...] = ab[:, :D]
    bx_ref[...] = ab[:, D:]


def _t_body(ax_ref, p_ref, sn_ref, t_ref, ps_ref, pq_ref):
    i = pl.program_id(0)
    t = (ax_ref[...] + p_ref[0] + p_ref[1]) * sn_ref[...]
    t_ref[...] = t
    s = jnp.sum(t.reshape(N_BLK // 8, 8, D), axis=0)
    q = jnp.sum((t * t).reshape(N_BLK // 8, 8, D), axis=0)

    @pl.when(i == 0)
    def _():
        ps_ref[...] = jnp.zeros_like(ps_ref)
        pq_ref[...] = jnp.zeros_like(pq_ref)

    ps_ref[...] += s
    pq_ref[...] += q


def _bn_body(t_ref, h_ref, ps_ref, pq_ref, g_ref, b_ref, o_ref, hs_ref):
    i = pl.program_id(0)
    mu = jnp.sum(ps_ref[...], axis=0, keepdims=True) * (1.0 / N)
    var = jnp.sum(pq_ref[...], axis=0, keepdims=True) * (1.0 / N) - mu * mu
    inv = lax.rsqrt(var + 1e-5)
    t = t_ref[...]
    hn = h_ref[...] + jnp.maximum(g_ref[...] * (t - mu) * inv + b_ref[...], 0.0)
    o_ref[...] = hn

    @pl.when(i == 0)
    def _():
        hs_ref[...] = jnp.zeros_like(hs_ref)

    hs_ref[...] += jnp.sum(hn.reshape(N_BLK // 8, 8, D), axis=0)


def _mlp_body(hs_ref, w0_ref, b0_ref, w1_ref, b1_ref, w2_ref, b2_ref, o_ref):
    m = jnp.sum(hs_ref[...], axis=0, keepdims=True) * (1.0 / N)
    y = jnp.maximum(
        jnp.dot(m, w0_ref[...], preferred_element_type=jnp.float32) + b0_ref[...], 0.0)
    y = jnp.maximum(
        jnp.dot(y, w1_ref[...], preferred_element_type=jnp.float32) + b1_ref[...], 0.0)
    o_ref[...] = (
        jnp.dot(y, w2_ref[...], preferred_element_type=jnp.float32) + b2_ref[...]
    )


_row_spec = pl.BlockSpec((N_BLK, D), lambda i: (i, 0))
_full_w = pl.BlockSpec((D, D), lambda i: (0, 0))
_full_b = pl.BlockSpec((1, D), lambda i: (0, 0))
_acc_spec = pl.BlockSpec((8, D), lambda i: (0, 0))


def _emb(x, w, b):
    return pl.pallas_call(
        _emb_body,
        grid=(GRID,),
        in_specs=[_row_spec, _full_w, _full_b],
        out_specs=_row_spec,
        out_shape=jax.ShapeDtypeStruct((N, D), jnp.float32),
    )(x, w, b)


def _ab(h, wcat, bcat):
    return pl.pallas_call(
        _ab_body,
        grid=(GRID,),
        in_specs=[
            _row_spec,
            pl.BlockSpec((D, 2 * D), lambda i: (0, 0)),
            pl.BlockSpec((1, 2 * D), lambda i: (0, 0)),
        ],
        out_specs=[_row_spec, _row_spec],
        out_shape=[
            jax.ShapeDtypeStruct((N, D), jnp.float32),
            jax.ShapeDtypeStruct((N, D), jnp.float32),
        ],
    )(h, wcat, bcat)


def _t_stage(ax, p, snorm):
    return pl.pallas_call(
        _t_body,
        grid=(GRID,),
        in_specs=[
            _row_spec,
            pl.BlockSpec((2, N_BLK, D), lambda i: (0, i, 0)),
            pl.BlockSpec((N_BLK, 1), lambda i: (i, 0)),
        ],
        out_specs=[_row_spec, _acc_spec, _acc_spec],
        out_shape=[
            jax.ShapeDtypeStruct((N, D), jnp.float32),
            jax.ShapeDtypeStruct((8, D), jnp.float32),
            jax.ShapeDtypeStruct((8, D), jnp.float32),
        ],
    )(ax, p, snorm)


def _bn_stage(t, h, ps, pq, g, b):
    return pl.pallas_call(
        _bn_body,
        grid=(GRID,),
        in_specs=[_row_spec, _row_spec, _acc_spec, _acc_spec, _full_b, _full_b],
        out_specs=[_row_spec, _acc_spec],
        out_shape=[
            jax.ShapeDtypeStruct((N, D), jnp.float32),
            jax.ShapeDtypeStruct((8, D), jnp.float32),
        ],
    )(t, h, ps, pq, g, b)


def _mlp(hs, w0, b0, w1, b1, w2, b2):
    return pl.pallas_call(
        _mlp_body,
        grid=(1,),
        in_specs=[_acc_spec, _full_w, _full_b, _full_w, _full_b, _full_w, _full_b],
        out_specs=pl.BlockSpec((1, D), lambda i: (0, 0)),
        out_shape=jax.ShapeDtypeStruct((1, D), jnp.float32),
    )(hs, w0, b0, w1, b1, w2, b2)


def _sc_agg_kernel(bx_hbm, src_hbm, dst_hbm, out_hbm,
                   src_v, dst_v, rows_v, zbuf_v, acc_sh, sem):
    cid = lax.axis_index("c")
    sid = lax.axis_index("s")

    @pl.loop(0, DCHUNK)
    def _(r):
        @pl.loop(0, D, step=16)
        def _(j):
            zbuf_v[r, pl.ds(j, 16)] = jnp.zeros((16,), jnp.float32)

    @pl.loop(DCHUNK * sid, N, step=DCHUNK * NS)
    def _(r):
        pltpu.sync_copy(zbuf_v, acc_sh.at[pl.ds(r, DCHUNK)])

    plsc.subcore_barrier()

    base = cid * (EDGES // NC) + sid * EPW

    @pl.loop(0, NCHUNK)
    def _(t):
        off = base + t * CHUNK
        pltpu.sync_copy(src_hbm.at[pl.ds(off, CHUNK)], src_v)
        pltpu.sync_copy(dst_hbm.at[pl.ds(off, CHUNK)], dst_v)
        pltpu.async_copy(bx_hbm.at[src_v], rows_v, sem).wait()
        pltpu.sync_copy(rows_v, acc_sh.at[dst_v], add=True)

    plsc.subcore_barrier()

    @pl.loop(DCHUNK * sid, N, step=DCHUNK * NS)
    def _(r):
        pltpu.sync_copy(acc_sh.at[pl.ds(r, DCHUNK)],
                        out_hbm.at[cid].at[pl.ds(r, DCHUNK)])


@jax.jit
def _sc_agg(bx, src, dst):
    mesh = plsc.VectorSubcoreMesh(core_axis_name="c", subcore_axis_name="s")
    f = pl.kernel(
        _sc_agg_kernel,
        mesh=mesh,
        out_type=jax.ShapeDtypeStruct((NC, N, D), jnp.float32),
        scratch_types=[
            pltpu.VMEM((CHUNK,), jnp.int32),
            pltpu.VMEM((CHUNK,), jnp.int32),
            pltpu.VMEM((CHUNK, D), jnp.float32),
            pltpu.VMEM((DCHUNK, D), jnp.float32),
            pltpu.VMEM_SHARED((N, D), jnp.float32),
            pltpu.SemaphoreType.DMA,
        ],
    )
    return f(bx, src, dst)


def kernel(X, E, snorm_n, snorm_e, params, edge_index):
    src = edge_index[0]
    dst = edge_index[1]

    H = _emb(X, params['emb_h_w'], params['emb_h_b'].reshape(1, D))

    for lp in params['layers']:
        wcat = jnp.concatenate([lp['A_w'], lp['B_w']], axis=1)
        bcat = jnp.concatenate([lp['A_b'], lp['B_b']]).reshape(1, 2 * D)
        ax, bx = _ab(H, wcat, bcat)
        p = _sc_agg(bx, src, dst)
        t, ps, pq = _t_stage(ax, p, snorm_n)
        H, hs = _bn_stage(t, H, ps, pq,
                          lp['bn_g'].reshape(1, D), lp['bn_b'].reshape(1, D))

    mlp = params['mlp']
    return _mlp(hs, mlp[0]['w'], mlp[0]['b'].reshape(1, D),
                mlp[1]['w'], mlp[1]['b'].reshape(1, D),
                mlp[2]['w'], mlp[2]['b'].reshape(1, D))


# speedup vs baseline: 10.1265x; 2.1502x over previous
import functools

import jax
import jax.numpy as jnp
from jax import lax
from jax.experimental import pallas as pl
from jax.experimental.pallas import tpu as pltpu
from jax.experimental.pallas import tpu_sc as plsc

N = 10000
D = 128
EDGES = 320000
N_BLK = 1000
GRID = N // N_BLK

NC = 2
NS = 16
CHUNK = 100
EPW = EDGES // (NC * NS)
GRP = 20
GROUPS = EPW // (GRP * CHUNK)
ZC = 80


def _emb_body(x_ref, w_ref, b_ref, o_ref):
    o_ref[...] = (
        jnp.dot(x_ref[...], w_ref[...], preferred_element_type=jnp.float32)
        + b_ref[...]
    )


def _ab_body(h_ref, w_ref, b_ref, ax_ref, bx_ref):
    ab = (
        jnp.dot(h_ref[...], w_ref[...], preferred_element_type=jnp.float32)
        + b_ref[...]
    )
    ax_ref[...] = ab[:, :D]
    bx_ref[...] = ab[:, D:]


def _t_body(ax_ref, p_ref, sn_ref, t_ref, ps_ref, pq_ref):
    i = pl.program_id(0)
    t = (ax_ref[...] + p_ref[0] + p_ref[1]) * sn_ref[...]
    t_ref[...] = t
    s = jnp.sum(t.reshape(N_BLK // 8, 8, D), axis=0)
    q = jnp.sum((t * t).reshape(N_BLK // 8, 8, D), axis=0)

    @pl.when(i == 0)
    def _():
        ps_ref[...] = jnp.zeros_like(ps_ref)
        pq_ref[...] = jnp.zeros_like(pq_ref)

    ps_ref[...] += s
    pq_ref[...] += q


def _bn_body(t_ref, h_ref, ps_ref, pq_ref, g_ref, b_ref, o_ref, hs_ref):
    i = pl.program_id(0)
    mu = jnp.sum(ps_ref[...], axis=0, keepdims=True) * (1.0 / N)
    var = jnp.sum(pq_ref[...], axis=0, keepdims=True) * (1.0 / N) - mu * mu
    inv = lax.rsqrt(var + 1e-5)
    t = t_ref[...]
    hn = h_ref[...] + jnp.maximum(g_ref[...] * (t - mu) * inv + b_ref[...], 0.0)
    o_ref[...] = hn

    @pl.when(i == 0)
    def _():
        hs_ref[...] = jnp.zeros_like(hs_ref)

    hs_ref[...] += jnp.sum(hn.reshape(N_BLK // 8, 8, D), axis=0)


def _mlp_body(hs_ref, w0_ref, b0_ref, w1_ref, b1_ref, w2_ref, b2_ref, o_ref):
    m = jnp.sum(hs_ref[...], axis=0, keepdims=True) * (1.0 / N)
    y = jnp.maximum(
        jnp.dot(m, w0_ref[...], preferred_element_type=jnp.float32) + b0_ref[...], 0.0)
    y = jnp.maximum(
        jnp.dot(y, w1_ref[...], preferred_element_type=jnp.float32) + b1_ref[...], 0.0)
    o_ref[...] = (
        jnp.dot(y, w2_ref[...], preferred_element_type=jnp.float32) + b2_ref[...]
    )


_row_spec = pl.BlockSpec((N_BLK, D), lambda i: (i, 0))
_full_w = pl.BlockSpec((D, D), lambda i: (0, 0))
_full_b = pl.BlockSpec((1, D), lambda i: (0, 0))
_acc_spec = pl.BlockSpec((8, D), lambda i: (0, 0))


def _emb(x, w, b):
    return pl.pallas_call(
        _emb_body,
        grid=(GRID,),
        in_specs=[_row_spec, _full_w, _full_b],
        out_specs=_row_spec,
        out_shape=jax.ShapeDtypeStruct((N, D), jnp.float32),
    )(x, w, b)


def _ab(h, wcat, bcat):
    return pl.pallas_call(
        _ab_body,
        grid=(GRID,),
        in_specs=[
            _row_spec,
            pl.BlockSpec((D, 2 * D), lambda i: (0, 0)),
            pl.BlockSpec((1, 2 * D), lambda i: (0, 0)),
        ],
        out_specs=[_row_spec, _row_spec],
        out_shape=[
            jax.ShapeDtypeStruct((N, D), jnp.float32),
            jax.ShapeDtypeStruct((N, D), jnp.float32),
        ],
    )(h, wcat, bcat)


def _t_stage(ax, p, snorm):
    return pl.pallas_call(
        _t_body,
        grid=(GRID,),
        in_specs=[
            _row_spec,
            pl.BlockSpec((2, N_BLK, D), lambda i: (0, i, 0)),
            pl.BlockSpec((N_BLK, 1), lambda i: (i, 0)),
        ],
        out_specs=[_row_spec, _acc_spec, _acc_spec],
        out_shape=[
            jax.ShapeDtypeStruct((N, D), jnp.float32),
            jax.ShapeDtypeStruct((8, D), jnp.float32),
            jax.ShapeDtypeStruct((8, D), jnp.float32),
        ],
    )(ax, p, snorm)


def _bn_stage(t, h, ps, pq, g, b):
    return pl.pallas_call(
        _bn_body,
        grid=(GRID,),
        in_specs=[_row_spec, _row_spec, _acc_spec, _acc_spec, _full_b, _full_b],
        out_specs=[_row_spec, _acc_spec],
        out_shape=[
            jax.ShapeDtypeStruct((N, D), jnp.float32),
            jax.ShapeDtypeStruct((8, D), jnp.float32),
        ],
    )(t, h, ps, pq, g, b)


def _mlp(hs, w0, b0, w1, b1, w2, b2):
    return pl.pallas_call(
        _mlp_body,
        grid=(1,),
        in_specs=[_acc_spec, _full_w, _full_b, _full_w, _full_b, _full_w, _full_b],
        out_specs=pl.BlockSpec((1, D), lambda i: (0, 0)),
        out_shape=jax.ShapeDtypeStruct((1, D), jnp.float32),
    )(hs, w0, b0, w1, b1, w2, b2)


def _sc_agg_kernel(bx_hbm, srcr_hbm, dstr_hbm, out_hbm,
                   sstage, dstage, rows_v, acc_sh,
                   g0, g1, isem):
    cid = lax.axis_index("c")
    sid = lax.axis_index("s")
    wid = cid * NS + sid
    gsems = (g0, g1)

    @pl.loop(0, ZC)
    def _(r):
        @pl.loop(0, D, step=16)
        def _(j):
            rows_v[0, r, pl.ds(j, 16)] = jnp.zeros((16,), jnp.float32)

    @pl.loop(ZC * sid, N, step=ZC * NS)
    def _(r):
        pltpu.sync_copy(rows_v.at[0].at[pl.ds(0, ZC)], acc_sh.at[pl.ds(r, ZC)])

    plsc.subcore_barrier()

    @pl.loop(0, GROUPS)
    def _(g):
        pltpu.sync_copy(srcr_hbm.at[wid, g], sstage)
        pltpu.sync_copy(dstr_hbm.at[wid, g], dstage)
        copies = [None] * GRP
        copies[0] = pltpu.async_copy(bx_hbm.at[sstage.at[0]], rows_v.at[0], g0)
        for b in range(GRP):
            if b + 1 < GRP:
                copies[b + 1] = pltpu.async_copy(
                    bx_hbm.at[sstage.at[b + 1]],
                    rows_v.at[(b + 1) % 2], gsems[(b + 1) % 2])
            copies[b].wait()
            pltpu.sync_copy(rows_v.at[b % 2], acc_sh.at[dstage.at[b]], add=True)

    plsc.subcore_barrier()

    @pl.loop(ZC * sid, N, step=ZC * NS)
    def _(r):
        pltpu.sync_copy(acc_sh.at[pl.ds(r, ZC)],
                        out_hbm.at[cid].at[pl.ds(r, ZC)])


@jax.jit
def _sc_agg(bx, src, dst):
    mesh = plsc.VectorSubcoreMesh(core_axis_name="c", subcore_axis_name="s")
    f = pl.kernel(
        _sc_agg_kernel,
        mesh=mesh,
        out_type=jax.ShapeDtypeStruct((NC, N, D), jnp.float32),
        scratch_types=[
            pltpu.VMEM((GRP, CHUNK), jnp.int32),
            pltpu.VMEM((GRP, CHUNK), jnp.int32),
            pltpu.VMEM((2, CHUNK, D), jnp.float32),
            pltpu.VMEM_SHARED((N, D), jnp.float32),
            pltpu.SemaphoreType.DMA,
            pltpu.SemaphoreType.DMA,
            pltpu.SemaphoreType.DMA,
        ],
    )
    return f(bx, src.reshape(NC * NS, GROUPS, GRP, CHUNK),
             dst.reshape(NC * NS, GROUPS, GRP, CHUNK))


def kernel(X, E, snorm_n, snorm_e, params, edge_index):
    src = edge_index[0]
    dst = edge_index[1]

    H = _emb(X, params['emb_h_w'], params['emb_h_b'].reshape(1, D))

    for lp in params['layers']:
        wcat = jnp.concatenate([lp['A_w'], lp['B_w']], axis=1)
        bcat = jnp.concatenate([lp['A_b'], lp['B_b']]).reshape(1, 2 * D)
        ax, bx = _ab(H, wcat, bcat)
        p = _sc_agg(bx, src, dst)
        t, ps, pq = _t_stage(ax, p, snorm_n)
        H, hs = _bn_stage(t, H, ps, pq,
                          lp['bn_g'].reshape(1, D), lp['bn_b'].reshape(1, D))

    mlp = params['mlp']
    return _mlp(hs, mlp[0]['w'], mlp[0]['b'].reshape(1, D),
                mlp[1]['w'], mlp[1]['b'].reshape(1, D),
                mlp[2]['w'], mlp[2]['b'].reshape(1, D))


# speedup vs baseline: 10.4135x; 1.0283x over previous
import functools

import jax
import jax.numpy as jnp
from jax import lax
from jax.experimental import pallas as pl
from jax.experimental.pallas import tpu as pltpu
from jax.experimental.pallas import tpu_sc as plsc

N = 10000
D = 128
EDGES = 320000
N_BLK = 1000
GRID = N // N_BLK

NC = 2
NS = 16
CHUNK = 125
EPW = EDGES // (NC * NS)
GRP = 16
GROUPS = EPW // (GRP * CHUNK)
ZC = 80


def _emb_body(x_ref, w_ref, b_ref, o_ref):
    o_ref[...] = (
        jnp.dot(x_ref[...], w_ref[...], preferred_element_type=jnp.float32)
        + b_ref[...]
    )


def _ab_body(h_ref, w_ref, b_ref, ax_ref, bx_ref):
    ab = (
        jnp.dot(h_ref[...], w_ref[...], preferred_element_type=jnp.float32)
        + b_ref[...]
    )
    ax_ref[...] = ab[:, :D]
    bx_ref[...] = ab[:, D:]


def _t_body(ax_ref, p_ref, sn_ref, t_ref, ps_ref, pq_ref):
    i = pl.program_id(0)
    t = (ax_ref[...] + p_ref[0] + p_ref[1]) * sn_ref[...]
    t_ref[...] = t
    s = jnp.sum(t.reshape(N_BLK // 8, 8, D), axis=0)
    q = jnp.sum((t * t).reshape(N_BLK // 8, 8, D), axis=0)

    @pl.when(i == 0)
    def _():
        ps_ref[...] = jnp.zeros_like(ps_ref)
        pq_ref[...] = jnp.zeros_like(pq_ref)

    ps_ref[...] += s
    pq_ref[...] += q


def _bn_body(t_ref, h_ref, ps_ref, pq_ref, g_ref, b_ref, o_ref, hs_ref):
    i = pl.program_id(0)
    mu = jnp.sum(ps_ref[...], axis=0, keepdims=True) * (1.0 / N)
    var = jnp.sum(pq_ref[...], axis=0, keepdims=True) * (1.0 / N) - mu * mu
    inv = lax.rsqrt(var + 1e-5)
    t = t_ref[...]
    hn = h_ref[...] + jnp.maximum(g_ref[...] * (t - mu) * inv + b_ref[...], 0.0)
    o_ref[...] = hn

    @pl.when(i == 0)
    def _():
        hs_ref[...] = jnp.zeros_like(hs_ref)

    hs_ref[...] += jnp.sum(hn.reshape(N_BLK // 8, 8, D), axis=0)


def _mlp_body(hs_ref, w0_ref, b0_ref, w1_ref, b1_ref, w2_ref, b2_ref, o_ref):
    m = jnp.sum(hs_ref[...], axis=0, keepdims=True) * (1.0 / N)
    y = jnp.maximum(
        jnp.dot(m, w0_ref[...], preferred_element_type=jnp.float32) + b0_ref[...], 0.0)
    y = jnp.maximum(
        jnp.dot(y, w1_ref[...], preferred_element_type=jnp.float32) + b1_ref[...], 0.0)
    o_ref[...] = (
        jnp.dot(y, w2_ref[...], preferred_element_type=jnp.float32) + b2_ref[...]
    )


_row_spec = pl.BlockSpec((N_BLK, D), lambda i: (i, 0))
_full_w = pl.BlockSpec((D, D), lambda i: (0, 0))
_full_b = pl.BlockSpec((1, D), lambda i: (0, 0))
_acc_spec = pl.BlockSpec((8, D), lambda i: (0, 0))


def _emb(x, w, b):
    return pl.pallas_call(
        _emb_body,
        grid=(GRID,),
        in_specs=[_row_spec, _full_w, _full_b],
        out_specs=_row_spec,
        out_shape=jax.ShapeDtypeStruct((N, D), jnp.float32),
    )(x, w, b)


def _ab(h, wcat, bcat):
    return pl.pallas_call(
        _ab_body,
        grid=(GRID,),
        in_specs=[
            _row_spec,
            pl.BlockSpec((D, 2 * D), lambda i: (0, 0)),
            pl.BlockSpec((1, 2 * D), lambda i: (0, 0)),
        ],
        out_specs=[_row_spec, _row_spec],
        out_shape=[
            jax.ShapeDtypeStruct((N, D), jnp.float32),
            jax.ShapeDtypeStruct((N, D), jnp.float32),
        ],
    )(h, wcat, bcat)


def _t_stage(ax, p, snorm):
    return pl.pallas_call(
        _t_body,
        grid=(GRID,),
        in_specs=[
            _row_spec,
            pl.BlockSpec((2, N_BLK, D), lambda i: (0, i, 0)),
            pl.BlockSpec((N_BLK, 1), lambda i: (i, 0)),
        ],
        out_specs=[_row_spec, _acc_spec, _acc_spec],
        out_shape=[
            jax.ShapeDtypeStruct((N, D), jnp.float32),
            jax.ShapeDtypeStruct((8, D), jnp.float32),
            jax.ShapeDtypeStruct((8, D), jnp.float32),
        ],
    )(ax, p, snorm)


def _bn_stage(t, h, ps, pq, g, b):
    return pl.pallas_call(
        _bn_body,
        grid=(GRID,),
        in_specs=[_row_spec, _row_spec, _acc_spec, _acc_spec, _full_b, _full_b],
        out_specs=[_row_spec, _acc_spec],
        out_shape=[
            jax.ShapeDtypeStruct((N, D), jnp.float32),
            jax.ShapeDtypeStruct((8, D), jnp.float32),
        ],
    )(t, h, ps, pq, g, b)


def _mlp(hs, w0, b0, w1, b1, w2, b2):
    return pl.pallas_call(
        _mlp_body,
        grid=(1,),
        in_specs=[_acc_spec, _full_w, _full_b, _full_w, _full_b, _full_w, _full_b],
        out_specs=pl.BlockSpec((1, D), lambda i: (0, 0)),
        out_shape=jax.ShapeDtypeStruct((1, D), jnp.float32),
    )(hs, w0, b0, w1, b1, w2, b2)


def _sc_agg_kernel(bx_hbm, srcr_hbm, dstr_hbm, out_hbm,
                   sstage, dstage, rows_v, acc_sh,
                   g0, g1, isem, zsem):
    cid = lax.axis_index("c")
    sid = lax.axis_index("s")
    wid = cid * NS + sid
    gsems = (g0, g1)

    cps = pltpu.async_copy(srcr_hbm.at[wid, 0], sstage, isem)
    cpd = pltpu.async_copy(dstr_hbm.at[wid, 0], dstage, isem)

    @pl.loop(0, ZC)
    def _(r):
        @pl.loop(0, D, step=16)
        def _(j):
            rows_v[0, r, pl.ds(j, 16)] = jnp.zeros((16,), jnp.float32)

    @pl.loop(ZC * sid, N, step=ZC * NS)
    def _(r):
        pltpu.async_copy(rows_v.at[0].at[pl.ds(0, ZC)],
                         acc_sh.at[pl.ds(r, ZC)], zsem)

    @pl.loop(ZC * sid, N, step=ZC * NS)
    def _(r):
        pltpu.make_async_copy(rows_v.at[0].at[pl.ds(0, ZC)],
                              acc_sh.at[pl.ds(r, ZC)], zsem).wait()

    cps.wait()
    cpd.wait()
    plsc.subcore_barrier()

    @pl.loop(0, GROUPS)
    def _(g):
        @pl.when(g > 0)
        def _():
            pltpu.sync_copy(srcr_hbm.at[wid, g], sstage)
            pltpu.sync_copy(dstr_hbm.at[wid, g], dstage)

        copies = [None] * GRP
        copies[0] = pltpu.async_copy(bx_hbm.at[sstage.at[0]], rows_v.at[0], g0)
        for b in range(GRP):
            if b + 1 < GRP:
                copies[b + 1] = pltpu.async_copy(
                    bx_hbm.at[sstage.at[b + 1]],
                    rows_v.at[(b + 1) % 2], gsems[(b + 1) % 2])
            copies[b].wait()
            pltpu.sync_copy(rows_v.at[b % 2], acc_sh.at[dstage.at[b]], add=True)

    plsc.subcore_barrier()

    @pl.loop(ZC * sid, N, step=ZC * NS)
    def _(r):
        pltpu.async_copy(acc_sh.at[pl.ds(r, ZC)],
                         out_hbm.at[cid].at[pl.ds(r, ZC)], zsem)

    @pl.loop(ZC * sid, N, step=ZC * NS)
    def _(r):
        pltpu.make_async_copy(acc_sh.at[pl.ds(r, ZC)],
                              out_hbm.at[cid].at[pl.ds(r, ZC)], zsem).wait()


@jax.jit
def _sc_agg(bx, src, dst):
    mesh = plsc.VectorSubcoreMesh(core_axis_name="c", subcore_axis_name="s")
    f = pl.kernel(
        _sc_agg_kernel,
        mesh=mesh,
        out_type=jax.ShapeDtypeStruct((NC, N, D), jnp.float32),
        scratch_types=[
            pltpu.VMEM((GRP, CHUNK), jnp.int32),
            pltpu.VMEM((GRP, CHUNK), jnp.int32),
            pltpu.VMEM((2, CHUNK, D), jnp.float32),
            pltpu.VMEM_SHARED((N, D), jnp.float32),
            pltpu.SemaphoreType.DMA,
            pltpu.SemaphoreType.DMA,
            pltpu.SemaphoreType.DMA,
            pltpu.SemaphoreType.DMA,
        ],
    )
    return f(bx, src.reshape(NC * NS, GROUPS, GRP, CHUNK),
             dst.reshape(NC * NS, GROUPS, GRP, CHUNK))


def kernel(X, E, snorm_n, snorm_e, params, edge_index):
    src = edge_index[0]
    dst = edge_index[1]

    H = _emb(X, params['emb_h_w'], params['emb_h_b'].reshape(1, D))

    for lp in params['layers']:
        wcat = jnp.concatenate([lp['A_w'], lp['B_w']], axis=1)
        bcat = jnp.concatenate([lp['A_b'], lp['B_b']]).reshape(1, 2 * D)
        ax, bx = _ab(H, wcat, bcat)
        p = _sc_agg(bx, src, dst)
        t, ps, pq = _t_stage(ax, p, snorm_n)
        H, hs = _bn_stage(t, H, ps, pq,
                          lp['bn_g'].reshape(1, D), lp['bn_b'].reshape(1, D))

    mlp = params['mlp']
    return _mlp(hs, mlp[0]['w'], mlp[0]['b'].reshape(1, D),
                mlp[1]['w'], mlp[1]['b'].reshape(1, D),
                mlp[2]['w'], mlp[2]['b'].reshape(1, D))


# speedup vs baseline: 11.5717x; 1.1112x over previous
import functools

import jax
import jax.numpy as jnp
from jax import lax
from jax.experimental import pallas as pl
from jax.experimental.pallas import tpu as pltpu
from jax.experimental.pallas import tpu_sc as plsc

N = 10000
D = 128
EDGES = 320000
N_BLK = 1000
GRID = N // N_BLK

NC = 2
NS = 16
CHUNK = 125
EPW = EDGES // (NC * NS)
GRP = 16
GROUPS = EPW // (GRP * CHUNK)
ZC = 80


def _emb_bx_body(x_ref, ew_ref, eb_ref, bw_ref, bb_ref, h_ref, bx_ref):
    h = jnp.dot(x_ref[...], ew_ref[...], preferred_element_type=jnp.float32) + eb_ref[...]
    h_ref[...] = h
    bx_ref[...] = jnp.dot(h, bw_ref[...], preferred_element_type=jnp.float32) + bb_ref[...]


def _phase_a(i, h_ref, p_ref, sn_ref, aw_ref, ab_ref, t_scr, h_scr, ps_scr, pq_scr):
    h = h_ref[...]
    t = (jnp.dot(h, aw_ref[...], preferred_element_type=jnp.float32) + ab_ref[...]
         + p_ref[0] + p_ref[1]) * sn_ref[...]
    t_scr[pl.ds(i * N_BLK, N_BLK), :] = t
    h_scr[pl.ds(i * N_BLK, N_BLK), :] = h
    s = jnp.sum(t.reshape(N_BLK // 8, 8, D), axis=0)
    q = jnp.sum((t * t).reshape(N_BLK // 8, 8, D), axis=0)

    @pl.when(i == 0)
    def _():
        ps_scr[...] = jnp.zeros_like(ps_scr)
        pq_scr[...] = jnp.zeros_like(pq_scr)

    ps_scr[...] += s
    pq_scr[...] += q


def _phase_b_hn(j, g_ref, b_ref, t_scr, h_scr, ps_scr, pq_scr):
    mu = jnp.sum(ps_scr[...], axis=0, keepdims=True) * (1.0 / N)
    var = jnp.sum(pq_scr[...], axis=0, keepdims=True) * (1.0 / N) - mu * mu
    inv = lax.rsqrt(var + 1e-5)
    t = t_scr[pl.ds(j * N_BLK, N_BLK), :]
    h = h_scr[pl.ds(j * N_BLK, N_BLK), :]
    return h + jnp.maximum(g_ref[...] * (t - mu) * inv + b_ref[...], 0.0)


def _mid_layer_body(h_ref, p_ref, sn_ref, aw_ref, ab_ref, g_ref, b_ref,
                    bw_ref, bb_ref, hn_ref, bxn_ref,
                    t_scr, h_scr, ps_scr, pq_scr):
    i = pl.program_id(0)

    @pl.when(i < GRID)
    def _():
        _phase_a(i, h_ref, p_ref, sn_ref, aw_ref, ab_ref,
                 t_scr, h_scr, ps_scr, pq_scr)

    @pl.when(i >= GRID)
    def _():
        hn = _phase_b_hn(i - GRID, g_ref, b_ref, t_scr, h_scr, ps_scr, pq_scr)
        hn_ref[...] = hn
        bxn_ref[...] = (
            jnp.dot(hn, bw_ref[...], preferred_element_type=jnp.float32)
            + bb_ref[...]
        )


def _last_layer_body(h_ref, p_ref, sn_ref, aw_ref, ab_ref, g_ref, b_ref,
                     hs_ref, t_scr, h_scr, ps_scr, pq_scr):
    i = pl.program_id(0)

    @pl.when(i < GRID)
    def _():
        _phase_a(i, h_ref, p_ref, sn_ref, aw_ref, ab_ref,
                 t_scr, h_scr, ps_scr, pq_scr)

    @pl.when(i >= GRID)
    def _():
        hn = _phase_b_hn(i - GRID, g_ref, b_ref, t_scr, h_scr, ps_scr, pq_scr)

        @pl.when(i == GRID)
        def _():
            hs_ref[...] = jnp.zeros_like(hs_ref)

        hs_ref[...] += jnp.sum(hn.reshape(N_BLK // 8, 8, D), axis=0)


def _mlp_body(hs_ref, w0_ref, b0_ref, w1_ref, b1_ref, w2_ref, b2_ref, o_ref):
    m = jnp.sum(hs_ref[...], axis=0, keepdims=True) * (1.0 / N)
    y = jnp.maximum(
        jnp.dot(m, w0_ref[...], preferred_element_type=jnp.float32) + b0_ref[...], 0.0)
    y = jnp.maximum(
        jnp.dot(y, w1_ref[...], preferred_element_type=jnp.float32) + b1_ref[...], 0.0)
    o_ref[...] = (
        jnp.dot(y, w2_ref[...], preferred_element_type=jnp.float32) + b2_ref[...]
    )


_row_spec = pl.BlockSpec((N_BLK, D), lambda i: (i, 0))
_full_w = pl.BlockSpec((D, D), lambda i: (0, 0))
_full_b = pl.BlockSpec((1, D), lambda i: (0, 0))
_acc_spec = pl.BlockSpec((8, D), lambda i: (0, 0))

_rowA_spec = pl.BlockSpec((N_BLK, D), lambda i: (jnp.where(i < GRID, i, GRID - 1), 0))
_pA_spec = pl.BlockSpec((2, N_BLK, D), lambda i: (0, jnp.where(i < GRID, i, GRID - 1), 0))
_snA_spec = pl.BlockSpec((N_BLK, 1), lambda i: (jnp.where(i < GRID, i, GRID - 1), 0))
_rowB_spec = pl.BlockSpec((N_BLK, D), lambda i: (jnp.maximum(i - GRID, 0), 0))
_accB_spec = pl.BlockSpec((8, D), lambda i: (0, 0))

_layer_scratch = [
    pltpu.VMEM((N, D), jnp.float32),
    pltpu.VMEM((N, D), jnp.float32),
    pltpu.VMEM((8, D), jnp.float32),
    pltpu.VMEM((8, D), jnp.float32),
]


def _emb_bx(x, ew, eb, bw, bb):
    return pl.pallas_call(
        _emb_bx_body,
        grid=(GRID,),
        in_specs=[_row_spec, _full_w, _full_b, _full_w, _full_b],
        out_specs=[_row_spec, _row_spec],
        out_shape=[
            jax.ShapeDtypeStruct((N, D), jnp.float32),
            jax.ShapeDtypeStruct((N, D), jnp.float32),
        ],
    )(x, ew, eb, bw, bb)


def _mid_layer(h, p, snorm, aw, ab, g, b, bw, bb):
    return pl.pallas_call(
        _mid_layer_body,
        grid=(2 * GRID,),
        in_specs=[_rowA_spec, _pA_spec, _snA_spec, _full_w, _full_b,
                  _full_b, _full_b, _full_w, _full_b],
        out_specs=[_rowB_spec, _rowB_spec],
        out_shape=[
            jax.ShapeDtypeStruct((N, D), jnp.float32),
            jax.ShapeDtypeStruct((N, D), jnp.float32),
        ],
        scratch_shapes=_layer_scratch,
    )(h, p, snorm, aw, ab, g, b, bw, bb)


def _last_layer(h, p, snorm, aw, ab, g, b):
    return pl.pallas_call(
        _last_layer_body,
        grid=(2 * GRID,),
        in_specs=[_rowA_spec, _pA_spec, _snA_spec, _full_w, _full_b,
                  _full_b, _full_b],
        out_specs=_accB_spec,
        out_shape=jax.ShapeDtypeStruct((8, D), jnp.float32),
        scratch_shapes=_layer_scratch,
    )(h, p, snorm, aw, ab, g, b)


def _mlp(hs, w0, b0, w1, b1, w2, b2):
    return pl.pallas_call(
        _mlp_body,
        grid=(1,),
        in_specs=[_acc_spec, _full_w, _full_b, _full_w, _full_b, _full_w, _full_b],
        out_specs=pl.BlockSpec((1, D), lambda i: (0, 0)),
        out_shape=jax.ShapeDtypeStruct((1, D), jnp.float32),
    )(hs, w0, b0, w1, b1, w2, b2)


def _sc_agg_kernel(bx_hbm, srcr_hbm, dstr_hbm, out_hbm,
                   sstage, dstage, rows_v, acc_sh,
                   g0, g1, isem, zsem):
    cid = lax.axis_index("c")
    sid = lax.axis_index("s")
    wid = cid * NS + sid
    gsems = (g0, g1)

    cps = pltpu.async_copy(srcr_hbm.at[wid, 0], sstage, isem)
    cpd = pltpu.async_copy(dstr_hbm.at[wid, 0], dstage, isem)

    @pl.loop(0, ZC)
    def _(r):
        @pl.loop(0, D, step=16)
        def _(j):
            rows_v[0, r, pl.ds(j, 16)] = jnp.zeros((16,), jnp.float32)

    @pl.loop(ZC * sid, N, step=ZC * NS)
    def _(r):
        pltpu.async_copy(rows_v.at[0].at[pl.ds(0, ZC)],
                         acc_sh.at[pl.ds(r, ZC)], zsem)

    @pl.loop(ZC * sid, N, step=ZC * NS)
    def _(r):
        pltpu.make_async_copy(rows_v.at[0].at[pl.ds(0, ZC)],
                              acc_sh.at[pl.ds(r, ZC)], zsem).wait()

    cps.wait()
    cpd.wait()
    plsc.subcore_barrier()

    @pl.loop(0, GROUPS)
    def _(g):
        @pl.when(g > 0)
        def _():
            pltpu.sync_copy(srcr_hbm.at[wid, g], sstage)
            pltpu.sync_copy(dstr_hbm.at[wid, g], dstage)

        copies = [None] * GRP
        copies[0] = pltpu.async_copy(bx_hbm.at[sstage.at[0]], rows_v.at[0], g0)
        for b in range(GRP):
            if b + 1 < GRP:
                copies[b + 1] = pltpu.async_copy(
                    bx_hbm.at[sstage.at[b + 1]],
                    rows_v.at[(b + 1) % 2], gsems[(b + 1) % 2])
            copies[b].wait()
            pltpu.sync_copy(rows_v.at[b % 2], acc_sh.at[dstage.at[b]], add=True)

    plsc.subcore_barrier()

    @pl.loop(ZC * sid, N, step=ZC * NS)
    def _(r):
        pltpu.async_copy(acc_sh.at[pl.ds(r, ZC)],
                         out_hbm.at[cid].at[pl.ds(r, ZC)], zsem)

    @pl.loop(ZC * sid, N, step=ZC * NS)
    def _(r):
        pltpu.make_async_copy(acc_sh.at[pl.ds(r, ZC)],
                              out_hbm.at[cid].at[pl.ds(r, ZC)], zsem).wait()


@jax.jit
def _sc_agg(bx, src, dst):
    mesh = plsc.VectorSubcoreMesh(core_axis_name="c", subcore_axis_name="s")
    f = pl.kernel(
        _sc_agg_kernel,
        mesh=mesh,
        out_type=jax.ShapeDtypeStruct((NC, N, D), jnp.float32),
        scratch_types=[
            pltpu.VMEM((GRP, CHUNK), jnp.int32),
            pltpu.VMEM((GRP, CHUNK), jnp.int32),
            pltpu.VMEM((2, CHUNK, D), jnp.float32),
            pltpu.VMEM_SHARED((N, D), jnp.float32),
            pltpu.SemaphoreType.DMA,
            pltpu.SemaphoreType.DMA,
            pltpu.SemaphoreType.DMA,
            pltpu.SemaphoreType.DMA,
        ],
    )
    return f(bx, src.reshape(NC * NS, GROUPS, GRP, CHUNK),
             dst.reshape(NC * NS, GROUPS, GRP, CHUNK))


def kernel(X, E, snorm_n, snorm_e, params, edge_index):
    src = edge_index[0]
    dst = edge_index[1]
    layers = params['layers']

    H, bx = _emb_bx(X, params['emb_h_w'], params['emb_h_b'].reshape(1, D),
                    layers[0]['B_w'], layers[0]['B_b'].reshape(1, D))

    for l, lp in enumerate(layers):
        p = _sc_agg(bx, src, dst)
        aw, ab = lp['A_w'], lp['A_b'].reshape(1, D)
        g, b = lp['bn_g'].reshape(1, D), lp['bn_b'].reshape(1, D)
        if l + 1 < len(layers):
            nxt = layers[l + 1]
            H, bx = _mid_layer(H, p, snorm_n, aw, ab, g, b,
                               nxt['B_w'], nxt['B_b'].reshape(1, D))
        else:
            hs = _last_layer(H, p, snorm_n, aw, ab, g, b)

    mlp = params['mlp']
    return _mlp(hs, mlp[0]['w'], mlp[0]['b'].reshape(1, D),
                mlp[1]['w'], mlp[1]['b'].reshape(1, D),
                mlp[2]['w'], mlp[2]['b'].reshape(1, D))


# speedup vs baseline: 12.1799x; 1.0526x over previous
import functools

import jax
import jax.numpy as jnp
from jax import lax
from jax.experimental import pallas as pl
from jax.experimental.pallas import tpu as pltpu
from jax.experimental.pallas import tpu_sc as plsc

N = 10000
D = 128
EDGES = 320000
N_BLK = 1000
GRID = N // N_BLK

NC = 2
NS = 16
CHUNK = 80
EPW = EDGES // (NC * NS)
GRP = 25
GROUPS = EPW // (GRP * CHUNK)
NB = 3
ZC = 80


def _emb_bx_body(x_ref, ew_ref, eb_ref, bw_ref, bb_ref, h_ref, bx_ref):
    h = jnp.dot(x_ref[...], ew_ref[...], preferred_element_type=jnp.float32) + eb_ref[...]
    h_ref[...] = h
    bx_ref[...] = jnp.dot(h, bw_ref[...], preferred_element_type=jnp.float32) + bb_ref[...]


def _phase_a(i, h_ref, p_ref, sn_ref, aw_ref, ab_ref, t_scr, h_scr, ps_scr, pq_scr):
    h = h_ref[...]
    t = (jnp.dot(h, aw_ref[...], preferred_element_type=jnp.float32) + ab_ref[...]
         + p_ref[0] + p_ref[1]) * sn_ref[...]
    t_scr[pl.ds(i * N_BLK, N_BLK), :] = t
    h_scr[pl.ds(i * N_BLK, N_BLK), :] = h
    s = jnp.sum(t.reshape(N_BLK // 8, 8, D), axis=0)
    q = jnp.sum((t * t).reshape(N_BLK // 8, 8, D), axis=0)

    @pl.when(i == 0)
    def _():
        ps_scr[...] = jnp.zeros_like(ps_scr)
        pq_scr[...] = jnp.zeros_like(pq_scr)

    ps_scr[...] += s
    pq_scr[...] += q


def _phase_b_hn(j, g_ref, b_ref, t_scr, h_scr, ps_scr, pq_scr):
    mu = jnp.sum(ps_scr[...], axis=0, keepdims=True) * (1.0 / N)
    var = jnp.sum(pq_scr[...], axis=0, keepdims=True) * (1.0 / N) - mu * mu
    inv = lax.rsqrt(var + 1e-5)
    t = t_scr[pl.ds(j * N_BLK, N_BLK), :]
    h = h_scr[pl.ds(j * N_BLK, N_BLK), :]
    return h + jnp.maximum(g_ref[...] * (t - mu) * inv + b_ref[...], 0.0)


def _mid_layer_body(h_ref, p_ref, sn_ref, aw_ref, ab_ref, g_ref, b_ref,
                    bw_ref, bb_ref, hn_ref, bxn_ref,
                    t_scr, h_scr, ps_scr, pq_scr):
    i = pl.program_id(0)

    @pl.when(i < GRID)
    def _():
        _phase_a(i, h_ref, p_ref, sn_ref, aw_ref, ab_ref,
                 t_scr, h_scr, ps_scr, pq_scr)

    @pl.when(i >= GRID)
    def _():
        hn = _phase_b_hn(i - GRID, g_ref, b_ref, t_scr, h_scr, ps_scr, pq_scr)
        hn_ref[...] = hn
        bxn_ref[...] = (
            jnp.dot(hn, bw_ref[...], preferred_element_type=jnp.float32)
            + bb_ref[...]
        )


def _last_layer_body(h_ref, p_ref, sn_ref, aw_ref, ab_ref, g_ref, b_ref,
                     hs_ref, t_scr, h_scr, ps_scr, pq_scr):
    i = pl.program_id(0)

    @pl.when(i < GRID)
    def _():
        _phase_a(i, h_ref, p_ref, sn_ref, aw_ref, ab_ref,
                 t_scr, h_scr, ps_scr, pq_scr)

    @pl.when(i >= GRID)
    def _():
        hn = _phase_b_hn(i - GRID, g_ref, b_ref, t_scr, h_scr, ps_scr, pq_scr)

        @pl.when(i == GRID)
        def _():
            hs_ref[...] = jnp.zeros_like(hs_ref)

        hs_ref[...] += jnp.sum(hn.reshape(N_BLK // 8, 8, D), axis=0)


def _mlp_body(hs_ref, w0_ref, b0_ref, w1_ref, b1_ref, w2_ref, b2_ref, o_ref):
    m = jnp.sum(hs_ref[...], axis=0, keepdims=True) * (1.0 / N)
    y = jnp.maximum(
        jnp.dot(m, w0_ref[...], preferred_element_type=jnp.float32) + b0_ref[...], 0.0)
    y = jnp.maximum(
        jnp.dot(y, w1_ref[...], preferred_element_type=jnp.float32) + b1_ref[...], 0.0)
    o_ref[...] = (
        jnp.dot(y, w2_ref[...], preferred_element_type=jnp.float32) + b2_ref[...]
    )


_row_spec = pl.BlockSpec((N_BLK, D), lambda i: (i, 0))
_full_w = pl.BlockSpec((D, D), lambda i: (0, 0))
_full_b = pl.BlockSpec((1, D), lambda i: (0, 0))
_acc_spec = pl.BlockSpec((8, D), lambda i: (0, 0))

_rowA_spec = pl.BlockSpec((N_BLK, D), lambda i: (jnp.where(i < GRID, i, GRID - 1), 0))
_pA_spec = pl.BlockSpec((2, N_BLK, D), lambda i: (0, jnp.where(i < GRID, i, GRID - 1), 0))
_snA_spec = pl.BlockSpec((N_BLK, 1), lambda i: (jnp.where(i < GRID, i, GRID - 1), 0))
_rowB_spec = pl.BlockSpec((N_BLK, D), lambda i: (jnp.maximum(i - GRID, 0), 0))
_accB_spec = pl.BlockSpec((8, D), lambda i: (0, 0))

_layer_scratch = [
    pltpu.VMEM((N, D), jnp.float32),
    pltpu.VMEM((N, D), jnp.float32),
    pltpu.VMEM((8, D), jnp.float32),
    pltpu.VMEM((8, D), jnp.float32),
]


def _emb_bx(x, ew, eb, bw, bb):
    return pl.pallas_call(
        _emb_bx_body,
        grid=(GRID,),
        in_specs=[_row_spec, _full_w, _full_b, _full_w, _full_b],
        out_specs=[_row_spec, _row_spec],
        out_shape=[
            jax.ShapeDtypeStruct((N, D), jnp.float32),
            jax.ShapeDtypeStruct((N, D), jnp.float32),
        ],
    )(x, ew, eb, bw, bb)


def _mid_layer(h, p, snorm, aw, ab, g, b, bw, bb):
    return pl.pallas_call(
        _mid_layer_body,
        grid=(2 * GRID,),
        in_specs=[_rowA_spec, _pA_spec, _snA_spec, _full_w, _full_b,
                  _full_b, _full_b, _full_w, _full_b],
        out_specs=[_rowB_spec, _rowB_spec],
        out_shape=[
            jax.ShapeDtypeStruct((N, D), jnp.float32),
            jax.ShapeDtypeStruct((N, D), jnp.float32),
        ],
        scratch_shapes=_layer_scratch,
    )(h, p, snorm, aw, ab, g, b, bw, bb)


def _last_layer(h, p, snorm, aw, ab, g, b):
    return pl.pallas_call(
        _last_layer_body,
        grid=(2 * GRID,),
        in_specs=[_rowA_spec, _pA_spec, _snA_spec, _full_w, _full_b,
                  _full_b, _full_b],
        out_specs=_accB_spec,
        out_shape=jax.ShapeDtypeStruct((8, D), jnp.float32),
        scratch_shapes=_layer_scratch,
    )(h, p, snorm, aw, ab, g, b)


def _mlp(hs, w0, b0, w1, b1, w2, b2):
    return pl.pallas_call(
        _mlp_body,
        grid=(1,),
        in_specs=[_acc_spec, _full_w, _full_b, _full_w, _full_b, _full_w, _full_b],
        out_specs=pl.BlockSpec((1, D), lambda i: (0, 0)),
        out_shape=jax.ShapeDtypeStruct((1, D), jnp.float32),
    )(hs, w0, b0, w1, b1, w2, b2)


def _sc_agg_kernel(bx_hbm, srcr_hbm, dstr_hbm, out_hbm,
                   sstage, dstage, rows_v, acc_sh,
                   g0, g1, g2, s0, s1, s2, isem, zsem):
    cid = lax.axis_index("c")
    sid = lax.axis_index("s")
    wid = cid * NS + sid
    gsems = (g0, g1, g2)
    ssems = (s0, s1, s2)

    cps = pltpu.async_copy(srcr_hbm.at[wid, 0], sstage, isem)
    cpd = pltpu.async_copy(dstr_hbm.at[wid, 0], dstage, isem)

    @pl.loop(0, ZC)
    def _(r):
        @pl.loop(0, D, step=16)
        def _(j):
            rows_v[0, r, pl.ds(j, 16)] = jnp.zeros((16,), jnp.float32)

    @pl.loop(ZC * sid, N, step=ZC * NS)
    def _(r):
        pltpu.async_copy(rows_v.at[0].at[pl.ds(0, ZC)],
                         acc_sh.at[pl.ds(r, ZC)], zsem)

    @pl.loop(ZC * sid, N, step=ZC * NS)
    def _(r):
        pltpu.make_async_copy(rows_v.at[0].at[pl.ds(0, ZC)],
                              acc_sh.at[pl.ds(r, ZC)], zsem).wait()

    cps.wait()
    cpd.wait()
    plsc.subcore_barrier()

    def scat_wait(slot):
        pltpu.make_async_copy(rows_v.at[slot], acc_sh.at[dstage.at[0]],
                              ssems[slot]).wait()

    @pl.loop(0, GROUPS)
    def _(g):
        @pl.when(g > 0)
        def _():
            for k in range(NB):
                scat_wait((GRP - NB + k) % NB)
            pltpu.sync_copy(srcr_hbm.at[wid, g], sstage)
            pltpu.sync_copy(dstr_hbm.at[wid, g], dstage)

        copies = [None] * GRP
        copies[0] = pltpu.async_copy(bx_hbm.at[sstage.at[0]], rows_v.at[0], g0)
        for b in range(GRP):
            if b + 1 < GRP:
                nb = (b + 1) % NB
                if b + 1 >= NB:
                    scat_wait(nb)
                copies[b + 1] = pltpu.async_copy(
                    bx_hbm.at[sstage.at[b + 1]], rows_v.at[nb], gsems[nb])
            copies[b].wait()
            pltpu.async_copy(rows_v.at[b % NB], acc_sh.at[dstage.at[b]],
                             ssems[b % NB], add=True)

    for k in range(NB):
        scat_wait((GRP - NB + k) % NB)

    plsc.subcore_barrier()

    @pl.loop(ZC * sid, N, step=ZC * NS)
    def _(r):
        pltpu.async_copy(acc_sh.at[pl.ds(r, ZC)],
                         out_hbm.at[cid].at[pl.ds(r, ZC)], zsem)

    @pl.loop(ZC * sid, N, step=ZC * NS)
    def _(r):
        pltpu.make_async_copy(acc_sh.at[pl.ds(r, ZC)],
                              out_hbm.at[cid].at[pl.ds(r, ZC)], zsem).wait()


@jax.jit
def _sc_agg(bx, src, dst):
    mesh = plsc.VectorSubcoreMesh(core_axis_name="c", subcore_axis_name="s")
    f = pl.kernel(
        _sc_agg_kernel,
        mesh=mesh,
        out_type=jax.ShapeDtypeStruct((NC, N, D), jnp.float32),
        scratch_types=[
            pltpu.VMEM((GRP, CHUNK), jnp.int32),
            pltpu.VMEM((GRP, CHUNK), jnp.int32),
            pltpu.VMEM((NB, CHUNK, D), jnp.float32),
            pltpu.VMEM_SHARED((N, D), jnp.float32),
            pltpu.SemaphoreType.DMA,
            pltpu.SemaphoreType.DMA,
            pltpu.SemaphoreType.DMA,
            pltpu.SemaphoreType.DMA,
            pltpu.SemaphoreType.DMA,
            pltpu.SemaphoreType.DMA,
            pltpu.SemaphoreType.DMA,
            pltpu.SemaphoreType.DMA,
        ],
    )
    return f(bx, src.reshape(NC * NS, GROUPS, GRP, CHUNK),
             dst.reshape(NC * NS, GROUPS, GRP, CHUNK))


def kernel(X, E, snorm_n, snorm_e, params, edge_index):
    src = edge_index[0]
    dst = edge_index[1]
    layers = params['layers']

    H, bx = _emb_bx(X, params['emb_h_w'], params['emb_h_b'].reshape(1, D),
                    layers[0]['B_w'], layers[0]['B_b'].reshape(1, D))

    for l, lp in enumerate(layers):
        p = _sc_agg(bx, src, dst)
        aw, ab = lp['A_w'], lp['A_b'].reshape(1, D)
        g, b = lp['bn_g'].reshape(1, D), lp['bn_b'].reshape(1, D)
        if l + 1 < len(layers):
            nxt = layers[l + 1]
            H, bx = _mid_layer(H, p, snorm_n, aw, ab, g, b,
                               nxt['B_w'], nxt['B_b'].reshape(1, D))
        else:
            hs = _last_layer(H, p, snorm_n, aw, ab, g, b)

    mlp = params['mlp']
    return _mlp(hs, mlp[0]['w'], mlp[0]['b'].reshape(1, D),
                mlp[1]['w'], mlp[1]['b'].reshape(1, D),
                mlp[2]['w'], mlp[2]['b'].reshape(1, D))


# speedup vs baseline: 12.2079x; 1.0023x over previous
import functools

import jax
import jax.numpy as jnp
from jax import lax
from jax.experimental import pallas as pl
from jax.experimental.pallas import tpu as pltpu
from jax.experimental.pallas import tpu_sc as plsc

N = 10000
D = 128
EDGES = 320000
N_BLK = 1000
GRID = N // N_BLK

NC = 2
NS = 16
CHUNK = 80
EPW = EDGES // (NC * NS)
GRP = 25
GROUPS = EPW // (GRP * CHUNK)
NB = 3
ZC = 80


def _emb_bx_body(x_ref, ew_ref, eb_ref, bw_ref, bb_ref, h_ref, bx_ref):
    h = jnp.dot(x_ref[...], ew_ref[...], preferred_element_type=jnp.float32) + eb_ref[...]
    h_ref[...] = h
    bx_ref[...] = jnp.dot(h, bw_ref[...], preferred_element_type=jnp.float32) + bb_ref[...]


def _phase_a(i, h_ref, p_ref, sn_ref, aw_ref, ab_ref, t_scr, h_scr, ps_scr, pq_scr):
    h = h_ref[...]
    t = (jnp.dot(h, aw_ref[...], preferred_element_type=jnp.float32) + ab_ref[...]
         + p_ref[0] + p_ref[1]) * sn_ref[...]
    t_scr[pl.ds(i * N_BLK, N_BLK), :] = t
    h_scr[pl.ds(i * N_BLK, N_BLK), :] = h
    s = jnp.sum(t.reshape(N_BLK // 8, 8, D), axis=0)
    q = jnp.sum((t * t).reshape(N_BLK // 8, 8, D), axis=0)

    @pl.when(i == 0)
    def _():
        ps_scr[...] = jnp.zeros_like(ps_scr)
        pq_scr[...] = jnp.zeros_like(pq_scr)

    ps_scr[...] += s
    pq_scr[...] += q


def _phase_b_hn(j, g_ref, b_ref, t_scr, h_scr, ps_scr, pq_scr):
    mu = jnp.sum(ps_scr[...], axis=0, keepdims=True) * (1.0 / N)
    var = jnp.sum(pq_scr[...], axis=0, keepdims=True) * (1.0 / N) - mu * mu
    inv = lax.rsqrt(var + 1e-5)
    t = t_scr[pl.ds(j * N_BLK, N_BLK), :]
    h = h_scr[pl.ds(j * N_BLK, N_BLK), :]
    return h + jnp.maximum(g_ref[...] * (t - mu) * inv + b_ref[...], 0.0)


def _mid_layer_body(h_ref, p_ref, sn_ref, aw_ref, ab_ref, g_ref, b_ref,
                    bw_ref, bb_ref, hn_ref, bxn_ref,
                    t_scr, h_scr, ps_scr, pq_scr):
    i = pl.program_id(0)

    @pl.when(i < GRID)
    def _():
        _phase_a(i, h_ref, p_ref, sn_ref, aw_ref, ab_ref,
                 t_scr, h_scr, ps_scr, pq_scr)

    @pl.when(i >= GRID)
    def _():
        hn = _phase_b_hn(i - GRID, g_ref, b_ref, t_scr, h_scr, ps_scr, pq_scr)
        hn_ref[...] = hn
        bxn_ref[...] = (
            jnp.dot(hn, bw_ref[...], preferred_element_type=jnp.float32)
            + bb_ref[...]
        )


def _last_layer_body(h_ref, p_ref, sn_ref, aw_ref, ab_ref, g_ref, b_ref,
                     w0_ref, b0_ref, w1_ref, b1_ref, w2_ref, b2_ref,
                     y_ref, t_scr, h_scr, ps_scr, pq_scr, hs_scr):
    i = pl.program_id(0)

    @pl.when(i < GRID)
    def _():
        _phase_a(i, h_ref, p_ref, sn_ref, aw_ref, ab_ref,
                 t_scr, h_scr, ps_scr, pq_scr)

    @pl.when(i >= GRID)
    def _():
        hn = _phase_b_hn(i - GRID, g_ref, b_ref, t_scr, h_scr, ps_scr, pq_scr)

        @pl.when(i == GRID)
        def _():
            hs_scr[...] = jnp.zeros_like(hs_scr)

        hs_scr[...] += jnp.sum(hn.reshape(N_BLK // 8, 8, D), axis=0)

    @pl.when(i == 2 * GRID - 1)
    def _():
        m = jnp.sum(hs_scr[...], axis=0, keepdims=True) * (1.0 / N)
        y = jnp.maximum(
            jnp.dot(m, w0_ref[...], preferred_element_type=jnp.float32)
            + b0_ref[...], 0.0)
        y = jnp.maximum(
            jnp.dot(y, w1_ref[...], preferred_element_type=jnp.float32)
            + b1_ref[...], 0.0)
        y_ref[...] = (
            jnp.dot(y, w2_ref[...], preferred_element_type=jnp.float32)
            + b2_ref[...]
        )


_row_spec = pl.BlockSpec((N_BLK, D), lambda i: (i, 0))
_full_w = pl.BlockSpec((D, D), lambda i: (0, 0))
_full_b = pl.BlockSpec((1, D), lambda i: (0, 0))
_acc_spec = pl.BlockSpec((8, D), lambda i: (0, 0))

_rowA_spec = pl.BlockSpec((N_BLK, D), lambda i: (jnp.where(i < GRID, i, GRID - 1), 0))
_pA_spec = pl.BlockSpec((2, N_BLK, D), lambda i: (0, jnp.where(i < GRID, i, GRID - 1), 0))
_snA_spec = pl.BlockSpec((N_BLK, 1), lambda i: (jnp.where(i < GRID, i, GRID - 1), 0))
_rowB_spec = pl.BlockSpec((N_BLK, D), lambda i: (jnp.maximum(i - GRID, 0), 0))
_accB_spec = pl.BlockSpec((8, D), lambda i: (0, 0))

_layer_scratch = [
    pltpu.VMEM((N, D), jnp.float32),
    pltpu.VMEM((N, D), jnp.float32),
    pltpu.VMEM((8, D), jnp.float32),
    pltpu.VMEM((8, D), jnp.float32),
]


def _emb_bx(x, ew, eb, bw, bb):
    return pl.pallas_call(
        _emb_bx_body,
        grid=(GRID,),
        in_specs=[_row_spec, _full_w, _full_b, _full_w, _full_b],
        out_specs=[_row_spec, _row_spec],
        out_shape=[
            jax.ShapeDtypeStruct((N, D), jnp.float32),
            jax.ShapeDtypeStruct((N, D), jnp.float32),
        ],
    )(x, ew, eb, bw, bb)


def _mid_layer(h, p, snorm, aw, ab, g, b, bw, bb):
    return pl.pallas_call(
        _mid_layer_body,
        grid=(2 * GRID,),
        in_specs=[_rowA_spec, _pA_spec, _snA_spec, _full_w, _full_b,
                  _full_b, _full_b, _full_w, _full_b],
        out_specs=[_rowB_spec, _rowB_spec],
        out_shape=[
            jax.ShapeDtypeStruct((N, D), jnp.float32),
            jax.ShapeDtypeStruct((N, D), jnp.float32),
        ],
        scratch_shapes=_layer_scratch,
    )(h, p, snorm, aw, ab, g, b, bw, bb)


def _last_layer(h, p, snorm, aw, ab, g, b, w0, b0, w1, b1, w2, b2):
    return pl.pallas_call(
        _last_layer_body,
        grid=(2 * GRID,),
        in_specs=[_rowA_spec, _pA_spec, _snA_spec, _full_w, _full_b,
                  _full_b, _full_b,
                  _full_w, _full_b, _full_w, _full_b, _full_w, _full_b],
        out_specs=pl.BlockSpec((1, D), lambda i: (0, 0)),
        out_shape=jax.ShapeDtypeStruct((1, D), jnp.float32),
        scratch_shapes=_layer_scratch + [pltpu.VMEM((8, D), jnp.float32)],
    )(h, p, snorm, aw, ab, g, b, w0, b0, w1, b1, w2, b2)


def _sc_agg_kernel(bx_hbm, srcr_hbm, dstr_hbm, out_hbm,
                   sstage, dstage, rows_v, acc_sh,
                   g0, g1, g2, s0, s1, s2, isem, zsem):
    cid = lax.axis_index("c")
    sid = lax.axis_index("s")
    wid = cid * NS + sid
    gsems = (g0, g1, g2)
    ssems = (s0, s1, s2)

    cps = pltpu.async_copy(srcr_hbm.at[wid, 0], sstage, isem)
    cpd = pltpu.async_copy(dstr_hbm.at[wid, 0], dstage, isem)

    @pl.loop(0, ZC)
    def _(r):
        @pl.loop(0, D, step=16)
        def _(j):
            rows_v[0, r, pl.ds(j, 16)] = jnp.zeros((16,), jnp.float32)

    @pl.loop(ZC * sid, N, step=ZC * NS)
    def _(r):
        pltpu.async_copy(rows_v.at[0].at[pl.ds(0, ZC)],
                         acc_sh.at[pl.ds(r, ZC)], zsem)

    @pl.loop(ZC * sid, N, step=ZC * NS)
    def _(r):
        pltpu.make_async_copy(rows_v.at[0].at[pl.ds(0, ZC)],
                              acc_sh.at[pl.ds(r, ZC)], zsem).wait()

    cps.wait()
    cpd.wait()
    plsc.subcore_barrier()

    def scat_wait(slot):
        pltpu.make_async_copy(rows_v.at[slot], acc_sh.at[dstage.at[0]],
                              ssems[slot]).wait()

    @pl.loop(0, GROUPS)
    def _(g):
        @pl.when(g > 0)
        def _():
            for k in range(NB):
                scat_wait((GRP - NB + k) % NB)
            pltpu.sync_copy(srcr_hbm.at[wid, g], sstage)
            pltpu.sync_copy(dstr_hbm.at[wid, g], dstage)

        copies = [None] * GRP
        copies[0] = pltpu.async_copy(bx_hbm.at[sstage.at[0]], rows_v.at[0], g0)
        for b in range(GRP):
            if b + 1 < GRP:
                nb = (b + 1) % NB
                if b + 1 >= NB:
                    scat_wait(nb)
                copies[b + 1] = pltpu.async_copy(
                    bx_hbm.at[sstage.at[b + 1]], rows_v.at[nb], gsems[nb])
            copies[b].wait()
            pltpu.async_copy(rows_v.at[b % NB], acc_sh.at[dstage.at[b]],
                             ssems[b % NB], add=True)

    for k in range(NB):
        scat_wait((GRP - NB + k) % NB)

    plsc.subcore_barrier()

    @pl.loop(ZC * sid, N, step=ZC * NS)
    def _(r):
        pltpu.async_copy(acc_sh.at[pl.ds(r, ZC)],
                         out_hbm.at[cid].at[pl.ds(r, ZC)], zsem)

    @pl.loop(ZC * sid, N, step=ZC * NS)
    def _(r):
        pltpu.make_async_copy(acc_sh.at[pl.ds(r, ZC)],
                              out_hbm.at[cid].at[pl.ds(r, ZC)], zsem).wait()


@jax.jit
def _sc_agg(bx, src, dst):
    mesh = plsc.VectorSubcoreMesh(core_axis_name="c", subcore_axis_name="s")
    f = pl.kernel(
        _sc_agg_kernel,
        mesh=mesh,
        out_type=jax.ShapeDtypeStruct((NC, N, D), jnp.float32),
        scratch_types=[
            pltpu.VMEM((GRP, CHUNK), jnp.int32),
            pltpu.VMEM((GRP, CHUNK), jnp.int32),
            pltpu.VMEM((NB, CHUNK, D), jnp.float32),
            pltpu.VMEM_SHARED((N, D), jnp.float32),
            pltpu.SemaphoreType.DMA,
            pltpu.SemaphoreType.DMA,
            pltpu.SemaphoreType.DMA,
            pltpu.SemaphoreType.DMA,
            pltpu.SemaphoreType.DMA,
            pltpu.SemaphoreType.DMA,
            pltpu.SemaphoreType.DMA,
            pltpu.SemaphoreType.DMA,
        ],
    )
    return f(bx, src.reshape(NC * NS, GROUPS, GRP, CHUNK),
             dst.reshape(NC * NS, GROUPS, GRP, CHUNK))


def kernel(X, E, snorm_n, snorm_e, params, edge_index):
    src = edge_index[0]
    dst = edge_index[1]
    layers = params['layers']

    H, bx = _emb_bx(X, params['emb_h_w'], params['emb_h_b'].reshape(1, D),
                    layers[0]['B_w'], layers[0]['B_b'].reshape(1, D))

    for l, lp in enumerate(layers):
        p = _sc_agg(bx, src, dst)
        aw, ab = lp['A_w'], lp['A_b'].reshape(1, D)
        g, b = lp['bn_g'].reshape(1, D), lp['bn_b'].reshape(1, D)
        if l + 1 < len(layers):
            nxt = layers[l + 1]
            H, bx = _mid_layer(H, p, snorm_n, aw, ab, g, b,
                               nxt['B_w'], nxt['B_b'].reshape(1, D))
        else:
            mlp = params['mlp']
            y = _last_layer(H, p, snorm_n, aw, ab, g, b,
                            mlp[0]['w'], mlp[0]['b'].reshape(1, D),
                            mlp[1]['w'], mlp[1]['b'].reshape(1, D),
                            mlp[2]['w'], mlp[2]['b'].reshape(1, D))

    return y
